# baseline (speedup 1.0000x reference)
import functools

import jax
import jax.numpy as jnp
from jax import lax
from jax.experimental import pallas as pl
from jax.experimental.pallas import tpu as pltpu
from jax.experimental.pallas import tpu_sc as plsc

_D = 64
_NW = 32
_CH = 128


@functools.lru_cache(maxsize=None)
def _make_gather(B: int):
    npw = B // _NW
    nch = npw // _CH
    mesh = plsc.VectorSubcoreMesh(core_axis_name="c", subcore_axis_name="s")

    @functools.partial(
        pl.kernel,
        mesh=mesh,
        compiler_params=pltpu.CompilerParams(use_tc_tiling_on_sc=False),
        out_type=jax.ShapeDtypeStruct((B, _D), jnp.float32),
        scratch_types=[
            pltpu.VMEM((nch, _CH), jnp.int32),
            pltpu.VMEM((2, _CH, _D), jnp.float32),
            pltpu.SemaphoreType.DMA,
        ],
    )
    def k(table_hbm, idx_hbm, out_hbm, idx_v, buf, gsem):
        c = lax.axis_index("c")
        s = lax.axis_index("s")
        wid = s * 2 + c
        base = wid * npw
        pltpu.sync_copy(idx_hbm.at[wid], idx_v)

        def body(j, carry):
            pltpu.async_copy(table_hbm.at[idx_v.at[j]], buf.at[0], gsem).wait()
            pltpu.sync_copy(buf.at[0], out_hbm.at[pl.ds(base + j * _CH, _CH)])
            return carry

        lax.fori_loop(0, nch, body, 0)

    return k


def kernel(indices, weight):
    shp = indices.shape
    B = indices.size
    idx3 = indices.astype(jnp.int32).reshape(_NW, B // _NW // _CH, _CH)
    out = _make_gather(B)(weight, idx3)
    return out.reshape(shp + (_D,))


# speedup vs baseline: 1.1136x; 1.1136x over previous
import functools

import jax
import jax.numpy as jnp
from jax import lax
from jax.experimental import pallas as pl
from jax.experimental.pallas import tpu as pltpu
from jax.experimental.pallas import tpu_sc as plsc

_D = 64
_NW = 32
_CH = 128
_GRP = 4


@functools.lru_cache(maxsize=None)
def _make_gather(B: int):
    npw = B // _NW
    nch = npw // _CH
    mesh = plsc.VectorSubcoreMesh(core_axis_name="c", subcore_axis_name="s")

    ngrp = nch // _GRP
    assert nch % _GRP == 0 and ngrp >= 3

    @functools.partial(
        pl.kernel,
        mesh=mesh,
        compiler_params=pltpu.CompilerParams(use_tc_tiling_on_sc=False),
        out_type=jax.ShapeDtypeStruct((B, _D), jnp.float32),
        scratch_types=[
            pltpu.VMEM((nch, _CH), jnp.int32),
            pltpu.VMEM((2 * _GRP, _CH, _D), jnp.float32),
            pltpu.SemaphoreType.DMA((2 * _GRP,)),
            pltpu.SemaphoreType.DMA((2 * _GRP,)),
        ],
    )
    def k(table_hbm, idx_hbm, out_hbm, idx_v, buf, gs, ws):
        c = lax.axis_index("c")
        s = lax.axis_index("s")
        wid = s * 2 + c
        base = wid * npw
        pltpu.sync_copy(idx_hbm.at[wid], idx_v)

        def gather(j, bb):
            pltpu.async_copy(table_hbm.at[idx_v.at[j]], buf.at[bb], gs.at[bb])

        def gwait(bb):
            pltpu.make_async_copy(
                table_hbm.at[idx_v.at[0]], buf.at[bb], gs.at[bb]).wait()

        def write(j, bb):
            pltpu.async_copy(
                buf.at[bb], out_hbm.at[pl.ds(base + j * _CH, _CH)], ws.at[bb])

        def wwait(bb):
            pltpu.make_async_copy(
                buf.at[bb], out_hbm.at[pl.ds(base, _CH)], ws.at[bb]).wait()

        for b in range(2 * _GRP):
            gather(b, b)

        def body(g, carry):
            bs = (g % 2) * _GRP
            for b in range(_GRP):
                gwait(bs + b)
                write(g * _GRP + b, bs + b)
            for b in range(_GRP):
                wwait(bs + b)
                gather((g + 2) * _GRP + b, bs + b)
            return carry

        lax.fori_loop(0, ngrp - 2, body, 0)

        for g in (ngrp - 2, ngrp - 1):
            bs = (g % 2) * _GRP
            for b in range(_GRP):
                gwait(bs + b)
                write(g * _GRP + b, bs + b)
            for b in range(_GRP):
                wwait(bs + b)

    return k


def kernel(indices, weight):
    shp = indices.shape
    B = indices.size
    idx3 = indices.astype(jnp.int32).reshape(_NW, B // _NW // _CH, _CH)
    out = _make_gather(B)(weight, idx3)
    return out.reshape(shp + (_D,))
